# baseline (speedup 1.0000x reference)
import functools

import jax
import jax.numpy as jnp
from jax import lax
from jax.experimental import pallas as pl
from jax.experimental.pallas import tpu as pltpu
from jax.experimental.pallas import tpu_sc as plsc

VOCAB = 50368
HID = 1024
EPS = 1e-5

NC = 2
NS = 16
NW = NC * NS
LANES = 16
VREGS_PER_ROW = HID // LANES

N_TOKENS = 4 * 4096
TOK_PER_W = N_TOKENS // NW
CHUNK = 32
N_CHUNKS = TOK_PER_W // CHUNK


def _lane_shuffle(v, perm):
    return lax.gather(
        v, perm[:, None],
        dimension_numbers=lax.GatherDimensionNumbers(
            offset_dims=(), collapsed_slice_dims=(0,), start_index_map=(0,)),
        slice_sizes=(1,),
        mode=lax.GatherScatterMode.PROMISE_IN_BOUNDS,
    )


def _rsqrt_newton(x):
    i = lax.bitcast_convert_type(x, jnp.int32)
    i = 0x5F3759DF - lax.shift_right_arithmetic(i, 1)
    y = lax.bitcast_convert_type(i, jnp.float32)
    for _ in range(3):
        y = y * (1.5 - 0.5 * x * y * y)
    return y


ROWQ = 8


def _tree_sum(xs):
    xs = list(xs)
    while len(xs) > 1:
        nxt = [a + b for a, b in zip(xs[0::2], xs[1::2])]
        if len(xs) % 2:
            nxt.append(xs[-1])
        xs = nxt
    return xs[0]


ROW_BLOCK = 16
JU = 8


NBUF = 3


def _layer_norm_chunk(rows_v, gamma_v, p_v, q_v):
    lane = lax.iota(jnp.int32, LANES)
    perms = [jnp.bitwise_xor(lane, k) for k in (8, 4, 2, 1)]

    def row_stats(t):
        s_parts, s2_parts = [], []
        for g8 in range(VREGS_PER_ROW // 8):
            vs = [rows_v[t, pl.ds((g8 * 8 + k) * LANES, LANES)]
                  for k in range(8)]
            s_parts.append(_tree_sum(vs))
            s2_parts.append(_tree_sum([v * v for v in vs]))
        return _tree_sum(s_parts), _tree_sum(s2_parts)

    def row_body(tp, _):
        stats = [row_stats(tp * ROWQ + u) for u in range(ROWQ)]
        for u, (s, s2) in enumerate(stats):
            t = tp * ROWQ + u
            for perm in perms:
                s = s + _lane_shuffle(s, perm)
                s2 = s2 + _lane_shuffle(s2, perm)
            mean = s * (1.0 / HID)
            var = s2 * (1.0 / HID) - mean * mean
            scale = _rsqrt_newton(var + EPS)
            p_v[t, pl.ds(0, LANES)] = scale
            q_v[t, pl.ds(0, LANES)] = mean * scale
        return 0

    lax.fori_loop(0, CHUNK // ROWQ, row_body, 0)

    def pb_body(rb, _):
        t0 = rb * ROW_BLOCK
        ps = [p_v[t0 + r, pl.ds(0, LANES)] for r in range(ROW_BLOCK)]
        qs = [q_v[t0 + r, pl.ds(0, LANES)] for r in range(ROW_BLOCK)]

        def j_body(j, _):
            for u in range(JU):
                jj = j * JU + u
                g = gamma_v[pl.ds(jj * LANES, LANES)]
                for r in range(ROW_BLOCK):
                    v = rows_v[t0 + r, pl.ds(jj * LANES, LANES)]
                    rows_v[t0 + r, pl.ds(jj * LANES, LANES)] = (
                        (v * ps[r] - qs[r]) * g)
            return 0

        lax.fori_loop(0, VREGS_PER_ROW // JU, j_body, 0)
        return 0

    lax.fori_loop(0, CHUNK // ROW_BLOCK, pb_body, 0)


def _ln_kernel(ids_hbm, table_hbm, gamma_hbm, out_hbm,
               idx_v, rows_v, gamma_v, p_v, q_v,
               gs0, gs1, gs2, os0, os1, os2):
    wid = lax.axis_index("s") * NC + lax.axis_index("c")
    base = wid * TOK_PER_W
    gsem = (gs0, gs1, gs2)
    osem = (os0, os1, os2)

    pltpu.sync_copy(gamma_hbm, gamma_v)
    pltpu.sync_copy(ids_hbm.at[pl.ds(base, TOK_PER_W)], idx_v)

    def gather_copy(c, s):
        return pltpu.make_async_copy(
            table_hbm.at[idx_v.at[pl.ds(c * CHUNK, CHUNK)]],
            rows_v.at[s], gsem[s])

    def out_copy(c, s):
        return pltpu.make_async_copy(
            rows_v.at[s], out_hbm.at[pl.ds(base + c * CHUNK, CHUNK)], osem[s])

    def chunk_step(c, s, pred):
        gather_copy(c, s).wait()
        sp = (s + 2) % NBUF

        def _prefetch():
            out_copy(c - 1, sp).wait()
            gather_copy(c + 2, sp).start()

        if pred is None:
            _prefetch()
        else:
            pl.when(pred)(_prefetch)
        _layer_norm_chunk(rows_v.at[s], gamma_v, p_v, q_v)
        out_copy(c, s).start()

    gather_copy(0, 0).start()
    gather_copy(1, 1).start()

    gather_copy(0, 0).wait()
    gather_copy(2, 2).start()
    _layer_norm_chunk(rows_v.at[0], gamma_v, p_v, q_v)
    out_copy(0, 0).start()

    def round_body(i, _):
        c0 = 3 * i
        chunk_step(c0 + 1, 1, None)
        chunk_step(c0 + 2, 2, c0 + 4 < N_CHUNKS)
        chunk_step(c0 + 3, 0, c0 + 5 < N_CHUNKS)
        return 0

    lax.fori_loop(0, (N_CHUNKS - 1) // 3, round_body, 0)

    out_copy(N_CHUNKS - 3, 1).wait()
    out_copy(N_CHUNKS - 2, 2).wait()
    out_copy(N_CHUNKS - 1, 0).wait()


@jax.jit
def kernel(input_ids, table, gamma):
    ids_flat = input_ids.reshape(-1).astype(jnp.int32)
    mesh = plsc.VectorSubcoreMesh(core_axis_name="c", subcore_axis_name="s")
    out = pl.kernel(
        _ln_kernel,
        out_type=jax.ShapeDtypeStruct((N_TOKENS, HID), jnp.float32),
        mesh=mesh,
        scratch_types=[
            pltpu.VMEM((TOK_PER_W,), jnp.int32),
            pltpu.VMEM((NBUF, CHUNK, HID), jnp.float32),
            pltpu.VMEM((HID,), jnp.float32),
            pltpu.VMEM((CHUNK, LANES), jnp.float32),
            pltpu.VMEM((CHUNK, LANES), jnp.float32),
            pltpu.SemaphoreType.DMA,
            pltpu.SemaphoreType.DMA,
            pltpu.SemaphoreType.DMA,
            pltpu.SemaphoreType.DMA,
            pltpu.SemaphoreType.DMA,
            pltpu.SemaphoreType.DMA,
        ],
    )(ids_flat, table, gamma)
    return out.reshape(input_ids.shape + (HID,))


# speedup vs baseline: 1.0027x; 1.0027x over previous
import functools

import jax
import jax.numpy as jnp
from jax import lax
from jax.experimental import pallas as pl
from jax.experimental.pallas import tpu as pltpu
from jax.experimental.pallas import tpu_sc as plsc

VOCAB = 50368
HID = 1024
EPS = 1e-5

NC = 2
NS = 16
NW = NC * NS
LANES = 16
VREGS_PER_ROW = HID // LANES

N_TOKENS = 4 * 4096
TOK_PER_W = N_TOKENS // NW
CHUNK = 32
N_CHUNKS = TOK_PER_W // CHUNK


def _lane_shuffle(v, perm):
    return lax.gather(
        v, perm[:, None],
        dimension_numbers=lax.GatherDimensionNumbers(
            offset_dims=(), collapsed_slice_dims=(0,), start_index_map=(0,)),
        slice_sizes=(1,),
        mode=lax.GatherScatterMode.PROMISE_IN_BOUNDS,
    )


def _rsqrt_newton(x):
    i = lax.bitcast_convert_type(x, jnp.int32)
    i = 0x5F3759DF - lax.shift_right_arithmetic(i, 1)
    y = lax.bitcast_convert_type(i, jnp.float32)
    for _ in range(3):
        y = y * (1.5 - 0.5 * x * y * y)
    return y


ROWQ = 4


def _tree_sum(xs):
    xs = list(xs)
    while len(xs) > 1:
        nxt = [a + b for a, b in zip(xs[0::2], xs[1::2])]
        if len(xs) % 2:
            nxt.append(xs[-1])
        xs = nxt
    return xs[0]


ROW_BLOCK = 16
JU = 8


NBUF = 3


def _layer_norm_chunk(rows_v, gamma_v, p_v, q_v):
    lane = lax.iota(jnp.int32, LANES)
    perms = [jnp.bitwise_xor(lane, k) for k in (8, 4, 2, 1)]

    def row_stats(t):
        s_parts, s2_parts = [], []
        for g8 in range(VREGS_PER_ROW // 8):
            vs = [rows_v[t, pl.ds((g8 * 8 + k) * LANES, LANES)]
                  for k in range(8)]
            s_parts.append(_tree_sum(vs))
            s2_parts.append(_tree_sum([v * v for v in vs]))
        return _tree_sum(s_parts), _tree_sum(s2_parts)

    def row_body(tp, _):
        stats = [row_stats(tp * ROWQ + u) for u in range(ROWQ)]
        for u, (s, s2) in enumerate(stats):
            t = tp * ROWQ + u
            p_v[t, pl.ds(0, LANES)] = s
            q_v[t, pl.ds(0, LANES)] = s2
        return 0

    lax.fori_loop(0, CHUNK // ROWQ, row_body, 0)

    def coef_body(tp, _):
        for u in range(8):
            t = tp * 8 + u
            s = p_v[t, pl.ds(0, LANES)]
            s2 = q_v[t, pl.ds(0, LANES)]
            for perm in perms:
                s = s + _lane_shuffle(s, perm)
                s2 = s2 + _lane_shuffle(s2, perm)
            mean = s * (1.0 / HID)
            var = s2 * (1.0 / HID) - mean * mean
            scale = _rsqrt_newton(var + EPS)
            p_v[t, pl.ds(0, LANES)] = scale
            q_v[t, pl.ds(0, LANES)] = mean * scale
        return 0

    lax.fori_loop(0, CHUNK // 8, coef_body, 0)

    def pb_body(rb, _):
        t0 = rb * ROW_BLOCK
        ps = [p_v[t0 + r, pl.ds(0, LANES)] for r in range(ROW_BLOCK)]
        qs = [q_v[t0 + r, pl.ds(0, LANES)] for r in range(ROW_BLOCK)]

        def j_body(j, _):
            for u in range(JU):
                jj = j * JU + u
                g = gamma_v[pl.ds(jj * LANES, LANES)]
                for r in range(ROW_BLOCK):
                    v = rows_v[t0 + r, pl.ds(jj * LANES, LANES)]
                    rows_v[t0 + r, pl.ds(jj * LANES, LANES)] = (
                        (v * ps[r] - qs[r]) * g)
            return 0

        lax.fori_loop(0, VREGS_PER_ROW // JU, j_body, 0)
        return 0

    lax.fori_loop(0, CHUNK // ROW_BLOCK, pb_body, 0)


def _ln_kernel(ids_hbm, table_hbm, gamma_hbm, out_hbm,
               idx_v, rows_v, gamma_v, p_v, q_v,
               gs0, gs1, gs2, os0, os1, os2):
    wid = lax.axis_index("s") * NC + lax.axis_index("c")
    base = wid * TOK_PER_W
    gsem = (gs0, gs1, gs2)
    osem = (os0, os1, os2)

    pltpu.sync_copy(gamma_hbm, gamma_v)
    pltpu.sync_copy(ids_hbm.at[pl.ds(base, TOK_PER_W)], idx_v)

    def gather_copy(c, s):
        return pltpu.make_async_copy(
            table_hbm.at[idx_v.at[pl.ds(c * CHUNK, CHUNK)]],
            rows_v.at[s], gsem[s])

    def out_copy(c, s):
        return pltpu.make_async_copy(
            rows_v.at[s], out_hbm.at[pl.ds(base + c * CHUNK, CHUNK)], osem[s])

    def chunk_step(c, s, pred):
        gather_copy(c, s).wait()
        sp = (s + 2) % NBUF

        def _prefetch():
            out_copy(c - 1, sp).wait()
            gather_copy(c + 2, sp).start()

        if pred is None:
            _prefetch()
        else:
            pl.when(pred)(_prefetch)
        _layer_norm_chunk(rows_v.at[s], gamma_v, p_v, q_v)
        out_copy(c, s).start()

    gather_copy(0, 0).start()
    gather_copy(1, 1).start()

    gather_copy(0, 0).wait()
    gather_copy(2, 2).start()
    _layer_norm_chunk(rows_v.at[0], gamma_v, p_v, q_v)
    out_copy(0, 0).start()

    def round_body(i, _):
        c0 = 3 * i
        chunk_step(c0 + 1, 1, None)
        chunk_step(c0 + 2, 2, c0 + 4 < N_CHUNKS)
        chunk_step(c0 + 3, 0, c0 + 5 < N_CHUNKS)
        return 0

    lax.fori_loop(0, (N_CHUNKS - 1) // 3, round_body, 0)

    out_copy(N_CHUNKS - 3, 1).wait()
    out_copy(N_CHUNKS - 2, 2).wait()
    out_copy(N_CHUNKS - 1, 0).wait()


@jax.jit
def kernel(input_ids, table, gamma):
    ids_flat = input_ids.reshape(-1).astype(jnp.int32)
    mesh = plsc.VectorSubcoreMesh(core_axis_name="c", subcore_axis_name="s")
    out = pl.kernel(
        _ln_kernel,
        out_type=jax.ShapeDtypeStruct((N_TOKENS, HID), jnp.float32),
        mesh=mesh,
        scratch_types=[
            pltpu.VMEM((TOK_PER_W,), jnp.int32),
            pltpu.VMEM((NBUF, CHUNK, HID), jnp.float32),
            pltpu.VMEM((HID,), jnp.float32),
            pltpu.VMEM((CHUNK, LANES), jnp.float32),
            pltpu.VMEM((CHUNK, LANES), jnp.float32),
            pltpu.SemaphoreType.DMA,
            pltpu.SemaphoreType.DMA,
            pltpu.SemaphoreType.DMA,
            pltpu.SemaphoreType.DMA,
            pltpu.SemaphoreType.DMA,
            pltpu.SemaphoreType.DMA,
        ],
    )(ids_flat, table, gamma)
    return out.reshape(input_ids.shape + (HID,))


# speedup vs baseline: 1.0200x; 1.0173x over previous
import functools

import jax
import jax.numpy as jnp
from jax import lax
from jax.experimental import pallas as pl
from jax.experimental.pallas import tpu as pltpu
from jax.experimental.pallas import tpu_sc as plsc

VOCAB = 50368
HID = 1024
EPS = 1e-5

NC = 2
NS = 16
NW = NC * NS
LANES = 16
VREGS_PER_ROW = HID // LANES

N_TOKENS = 4 * 4096
TOK_PER_W = N_TOKENS // NW
CHUNK = 32
N_CHUNKS = TOK_PER_W // CHUNK


def _lane_shuffle(v, perm):
    return lax.gather(
        v, perm[:, None],
        dimension_numbers=lax.GatherDimensionNumbers(
            offset_dims=(), collapsed_slice_dims=(0,), start_index_map=(0,)),
        slice_sizes=(1,),
        mode=lax.GatherScatterMode.PROMISE_IN_BOUNDS,
    )


def _rsqrt_newton(x):
    i = lax.bitcast_convert_type(x, jnp.int32)
    i = 0x5F3759DF - lax.shift_right_arithmetic(i, 1)
    y = lax.bitcast_convert_type(i, jnp.float32)
    for _ in range(3):
        y = y * (1.5 - 0.5 * x * y * y)
    return y


ROWQ = 4


def _tree_sum(xs):
    xs = list(xs)
    while len(xs) > 1:
        nxt = [a + b for a, b in zip(xs[0::2], xs[1::2])]
        if len(xs) % 2:
            nxt.append(xs[-1])
        xs = nxt
    return xs[0]


ROW_BLOCK = 16
JU = 8


NBUF = 3


def _layer_norm_chunk(rows_v, gamma_v, p_v, q_v):
    lane = lax.iota(jnp.int32, LANES)
    perms = [jnp.bitwise_xor(lane, k) for k in (8, 4, 2, 1)]

    def row_stats(t):
        s_parts, s2_parts = [], []
        for g8 in range(VREGS_PER_ROW // 8):
            vs = [rows_v[t, pl.ds((g8 * 8 + k) * LANES, LANES)]
                  for k in range(8)]
            s_parts.append(_tree_sum(vs))
            s2_parts.append(_tree_sum([v * v for v in vs]))
        return _tree_sum(s_parts), _tree_sum(s2_parts)

    def row_body(tp, _):
        stats = [row_stats(tp * ROWQ + u) for u in range(ROWQ)]
        for u, (s, s2) in enumerate(stats):
            t = tp * ROWQ + u
            for perm in perms:
                s = s + _lane_shuffle(s, perm)
                s2 = s2 + _lane_shuffle(s2, perm)
            mean = s * (1.0 / HID)
            var = s2 * (1.0 / HID) - mean * mean
            scale = _rsqrt_newton(var + EPS)
            p_v[t, pl.ds(0, LANES)] = scale
            q_v[t, pl.ds(0, LANES)] = mean * scale
        return 0

    lax.fori_loop(0, CHUNK // ROWQ, row_body, 0)

    def pb_body(rb, _):
        t0 = rb * ROW_BLOCK
        ps = [p_v[t0 + r, pl.ds(0, LANES)] for r in range(ROW_BLOCK)]
        qs = [q_v[t0 + r, pl.ds(0, LANES)] for r in range(ROW_BLOCK)]

        def j_body(j, _):
            for u in range(JU):
                jj = j * JU + u
                g = gamma_v[pl.ds(jj * LANES, LANES)]
                for r in range(ROW_BLOCK):
                    v = rows_v[t0 + r, pl.ds(jj * LANES, LANES)]
                    rows_v[t0 + r, pl.ds(jj * LANES, LANES)] = (
                        (v * ps[r] - qs[r]) * g)
            return 0

        lax.fori_loop(0, VREGS_PER_ROW // JU, j_body, 0)
        return 0

    lax.fori_loop(0, CHUNK // ROW_BLOCK, pb_body, 0)


def _ln_kernel(ids_hbm, table_hbm, gamma_hbm, out_hbm,
               idx_v, rows_v, gamma_v, p_v, q_v,
               gs0, gs1, gs2, os0, os1, os2):
    wid = lax.axis_index("s") * NC + lax.axis_index("c")
    base = wid * TOK_PER_W
    gsem = (gs0, gs1, gs2)
    osem = (os0, os1, os2)

    pltpu.sync_copy(gamma_hbm, gamma_v)
    pltpu.sync_copy(ids_hbm.at[pl.ds(base, TOK_PER_W)], idx_v)

    def gather_copy(c, s):
        return pltpu.make_async_copy(
            table_hbm.at[idx_v.at[pl.ds(c * CHUNK, CHUNK)]],
            rows_v.at[s], gsem[s])

    def out_copy(c, s):
        return pltpu.make_async_copy(
            rows_v.at[s], out_hbm.at[pl.ds(base + c * CHUNK, CHUNK)], osem[s])

    def chunk_step(c, s, pred):
        gather_copy(c, s).wait()
        sp = (s + 2) % NBUF

        def _prefetch():
            out_copy(c - 1, sp).wait()
            gather_copy(c + 2, sp).start()

        if pred is None:
            _prefetch()
        else:
            pl.when(pred)(_prefetch)
        _layer_norm_chunk(rows_v.at[s], gamma_v, p_v, q_v)
        out_copy(c, s).start()

    gather_copy(0, 0).start()
    gather_copy(1, 1).start()

    gather_copy(0, 0).wait()
    gather_copy(2, 2).start()
    _layer_norm_chunk(rows_v.at[0], gamma_v, p_v, q_v)
    out_copy(0, 0).start()

    def round_body(i, _):
        c0 = 3 * i
        chunk_step(c0 + 1, 1, None)
        chunk_step(c0 + 2, 2, c0 + 4 < N_CHUNKS)
        chunk_step(c0 + 3, 0, c0 + 5 < N_CHUNKS)
        return 0

    lax.fori_loop(0, (N_CHUNKS - 1) // 3, round_body, 0)

    out_copy(N_CHUNKS - 3, 1).wait()
    out_copy(N_CHUNKS - 2, 2).wait()
    out_copy(N_CHUNKS - 1, 0).wait()


@jax.jit
def kernel(input_ids, table, gamma):
    ids_flat = input_ids.reshape(-1).astype(jnp.int32)
    mesh = plsc.VectorSubcoreMesh(core_axis_name="c", subcore_axis_name="s")
    out = pl.kernel(
        _ln_kernel,
        out_type=jax.ShapeDtypeStruct((N_TOKENS, HID), jnp.float32),
        mesh=mesh,
        scratch_types=[
            pltpu.VMEM((TOK_PER_W,), jnp.int32),
            pltpu.VMEM((NBUF, CHUNK, HID), jnp.float32),
            pltpu.VMEM((HID,), jnp.float32),
            pltpu.VMEM((CHUNK, LANES), jnp.float32),
            pltpu.VMEM((CHUNK, LANES), jnp.float32),
            pltpu.SemaphoreType.DMA,
            pltpu.SemaphoreType.DMA,
            pltpu.SemaphoreType.DMA,
            pltpu.SemaphoreType.DMA,
            pltpu.SemaphoreType.DMA,
            pltpu.SemaphoreType.DMA,
        ],
    )(ids_flat, table, gamma)
    return out.reshape(input_ids.shape + (HID,))
